# Initial kernel scaffold; baseline (speedup 1.0000x reference)
import jax
import jax.numpy as jnp
from jax.experimental import pallas as pl


def kernel(x, edge_index, edge_weight, batch, W0, W1, W2, lin1_W, lin1_b, bn_gamma, bn_beta, lin2_W, lin2_b):
    raise NotImplementedError("write your pallas kernel here")


# speedup vs baseline: 10.7576x; 10.7576x over previous
import functools

import jax
import jax.numpy as jnp
from jax import lax
from jax.experimental import pallas as pl
from jax.experimental.pallas import tpu as pltpu
from jax.experimental.pallas import tpu_sc as plsc

F32 = jnp.float32
I32 = jnp.int32

NC = 2
NS = 16
NW = NC * NS

BM = 1024


def _pad_up(v, m):
    return (v + m - 1) // m * m


def _sc_mesh():
    return plsc.VectorSubcoreMesh(
        core_axis_name="c", subcore_axis_name="s", num_cores=NC, num_subcores=NS
    )


def _make_deg_kernel(npad, k):
    @functools.partial(
        pl.kernel,
        out_type=jax.ShapeDtypeStruct((NW, npad), F32),
        mesh=_sc_mesh(),
        scratch_types=[
            pltpu.VMEM((k, 128), I32),
            pltpu.VMEM((npad,), F32),
        ],
        compiler_params=pltpu.CompilerParams(needs_layout_passes=False),
    )
    def deg_kernel(dst_hbm, out_hbm, didx, dacc):
        cid = lax.axis_index("c")
        sid = lax.axis_index("s")
        wid = cid * NS + sid
        zero16 = jnp.zeros((16,), F32)

        def zbody(i, _):
            dacc[pl.ds(i * 16, 16)] = zero16
            return 0

        lax.fori_loop(0, npad // 16, zbody, 0)
        pltpu.sync_copy(dst_hbm.at[wid], didx)
        one16 = jnp.full((16,), 1.0, F32)

        def ebody(j, _):
            for i in range(8):
                idx = didx[j, pl.ds(i * 16, 16)]
                plsc.addupdate_scatter(dacc, [idx], one16)
            return 0

        lax.fori_loop(0, k, ebody, 0)
        pltpu.sync_copy(dacc, out_hbm.at[wid])

    return deg_kernel


def _make_agg_kernel(npad, k):
    rows_per_sub = npad // NS

    @functools.partial(
        pl.kernel,
        out_type=jax.ShapeDtypeStruct((NC, npad, 128), F32),
        mesh=_sc_mesh(),
        scratch_types=[
            pltpu.VMEM((k, 128), I32),
            pltpu.VMEM((k, 128), I32),
            pltpu.VMEM((128, 128), F32),
            pltpu.VMEM_SHARED((npad, 128), F32),
            pltpu.SemaphoreType.DMA,
        ],
    )
    def agg_kernel(y_hbm, src_hbm, dst_hbm, zeros_hbm, out_hbm, sidx, didx, rows, acc, sem):
        cid = lax.axis_index("c")
        sid = lax.axis_index("s")
        wid = cid * NS + sid
        pltpu.sync_copy(zeros_hbm, acc.at[pl.ds(sid * rows_per_sub, rows_per_sub)])
        pltpu.sync_copy(src_hbm.at[wid], sidx)
        pltpu.sync_copy(dst_hbm.at[wid], didx)
        plsc.subcore_barrier()

        def chunk(j, _):
            pltpu.async_copy(y_hbm.at[sidx.at[j]], rows, sem).wait()
            pltpu.sync_copy(rows, acc.at[didx.at[j]], add=True)
            return 0

        lax.fori_loop(0, k, chunk, 0)
        plsc.subcore_barrier()
        pltpu.sync_copy(
            acc.at[pl.ds(sid * rows_per_sub, rows_per_sub)],
            out_hbm.at[cid, pl.ds(sid * rows_per_sub, rows_per_sub)],
        )

    return agg_kernel


def _mm1_body(degp_ref, x_ref, w_ref, y_ref, dinv_ref):
    deg = jnp.sum(degp_ref[...], axis=0) + 1.0
    dinv = lax.rsqrt(deg)
    xw = jnp.dot(x_ref[...], w_ref[...], preferred_element_type=F32)
    y_ref[...] = xw * dinv[:, None]
    dinv_ref[...] = dinv


def _mmc_body(p_ref, y_ref, dinv_ref, w_ref, o_ref):
    dinv = dinv_ref[...][:, None]
    h = jnp.maximum((p_ref[0] + p_ref[1] + y_ref[...]) * dinv, 0.0)
    xw = jnp.dot(h, w_ref[...], preferred_element_type=F32)
    o_ref[...] = xw * dinv


def _pool_body(p_ref, y_ref, dinv_ref, batch_ref, gsum_ref, cnt_ref, *, G):
    i = pl.program_id(0)
    dinv = dinv_ref[...][:, None]
    h = jnp.maximum((p_ref[0] + p_ref[1] + y_ref[...]) * dinv, 0.0)
    b = batch_ref[...]
    onehot = (b[:, None] == lax.broadcasted_iota(I32, (1, G), 1)).astype(F32)
    gs = lax.dot_general(onehot, h, (((0,), (0,)), ((), ())),
                         preferred_element_type=F32,
                         precision=lax.Precision.HIGHEST)
    ones = jnp.ones((h.shape[0], 128), F32)
    cs = lax.dot_general(onehot, ones, (((0,), (0,)), ((), ())),
                         preferred_element_type=F32,
                         precision=lax.Precision.HIGHEST)

    @pl.when(i == 0)
    def _():
        gsum_ref[...] = gs
        cnt_ref[...] = cs

    @pl.when(i != 0)
    def _():
        gsum_ref[...] += gs
        cnt_ref[...] += cs


def _mlp_body(gsum_ref, cnt_ref, w1_ref, b1_ref, gam_ref, bet_ref, w2_ref, b2_ref,
              xlog_ref, xsig_ref, last_ref):
    cnt = jnp.maximum(cnt_ref[...], 1.0)
    g = gsum_ref[...] / cnt
    m = jnp.dot(g, w1_ref[...], preferred_element_type=F32) + b1_ref[...][None, :]
    mu = jnp.mean(m, axis=0, keepdims=True)
    var = jnp.mean((m - mu) * (m - mu), axis=0, keepdims=True)
    m = (m - mu) / jnp.sqrt(var + 1e-5) * gam_ref[...][None, :] + bet_ref[...][None, :]
    m = jnp.maximum(m, 0.0)
    out = jnp.dot(m, w2_ref[...], preferred_element_type=F32) + b2_ref[...][None, :]
    mx = jnp.max(out, axis=1, keepdims=True)
    lse = mx + jnp.log(jnp.sum(jnp.exp(out - mx), axis=1, keepdims=True))
    xlog_ref[...] = out - lse
    xsig_ref[...] = 1.0 / (1.0 + jnp.exp(-out))
    last_ref[...] = out


def kernel(x, edge_index, edge_weight, batch, W0, W1, W2,
           lin1_W, lin1_b, bn_gamma, bn_beta, lin2_W, lin2_b):
    del edge_weight
    n, d = x.shape
    e = edge_index.shape[1]
    h_dim = W0.shape[1]
    o_dim = lin2_W.shape[1]
    g_num = 128

    npad = _pad_up(n + 1, BM)
    ept = _pad_up(-(-e // NW), 128)
    k = ept // 128
    epad = NW * ept

    src = edge_index[0]
    dst = edge_index[1]
    pad_idx = jnp.full((epad - e,), n, I32)
    srcp = jnp.concatenate([src, pad_idx]).reshape(NW, k, 128)
    dstp = jnp.concatenate([dst, pad_idx]).reshape(NW, k, 128)
    xp = jnp.pad(x, ((0, npad - n), (0, 0)))
    batchp = jnp.concatenate([batch, jnp.full((npad - n,), g_num, I32)])
    zeros2d = jnp.zeros((npad // NS, h_dim), F32)

    deg_call = _make_deg_kernel(npad, k)
    agg_call = _make_agg_kernel(npad, k)

    grid = npad // BM
    mm1 = pl.pallas_call(
        _mm1_body,
        grid=(grid,),
        in_specs=[
            pl.BlockSpec((NW, BM), lambda i: (0, i)),
            pl.BlockSpec((BM, d), lambda i: (i, 0)),
            pl.BlockSpec((d, h_dim), lambda i: (0, 0)),
        ],
        out_specs=[
            pl.BlockSpec((BM, h_dim), lambda i: (i, 0)),
            pl.BlockSpec((BM,), lambda i: (i,)),
        ],
        out_shape=[
            jax.ShapeDtypeStruct((npad, h_dim), F32),
            jax.ShapeDtypeStruct((npad,), F32),
        ],
    )
    mmc = pl.pallas_call(
        _mmc_body,
        grid=(grid,),
        in_specs=[
            pl.BlockSpec((NC, BM, h_dim), lambda i: (0, i, 0)),
            pl.BlockSpec((BM, h_dim), lambda i: (i, 0)),
            pl.BlockSpec((BM,), lambda i: (i,)),
            pl.BlockSpec((h_dim, h_dim), lambda i: (0, 0)),
        ],
        out_specs=pl.BlockSpec((BM, h_dim), lambda i: (i, 0)),
        out_shape=jax.ShapeDtypeStruct((npad, h_dim), F32),
    )
    pool = pl.pallas_call(
        functools.partial(_pool_body, G=g_num),
        grid=(grid,),
        in_specs=[
            pl.BlockSpec((NC, BM, h_dim), lambda i: (0, i, 0)),
            pl.BlockSpec((BM, h_dim), lambda i: (i, 0)),
            pl.BlockSpec((BM,), lambda i: (i,)),
            pl.BlockSpec((BM,), lambda i: (i,)),
        ],
        out_specs=[
            pl.BlockSpec((g_num, h_dim), lambda i: (0, 0)),
            pl.BlockSpec((g_num, 128), lambda i: (0, 0)),
        ],
        out_shape=[
            jax.ShapeDtypeStruct((g_num, h_dim), F32),
            jax.ShapeDtypeStruct((g_num, 128), F32),
        ],
    )
    mlp = pl.pallas_call(
        _mlp_body,
        out_shape=[
            jax.ShapeDtypeStruct((g_num, o_dim), F32),
            jax.ShapeDtypeStruct((g_num, o_dim), F32),
            jax.ShapeDtypeStruct((g_num, o_dim), F32),
        ],
    )

    degp = deg_call(dstp)
    y0, dinv = mm1(degp, xp, W0)
    p0 = agg_call(y0, srcp, dstp, zeros2d)
    y1 = mmc(p0, y0, dinv, W1)
    p1 = agg_call(y1, srcp, dstp, zeros2d)
    y2 = mmc(p1, y1, dinv, W2)
    p2 = agg_call(y2, srcp, dstp, zeros2d)
    gsum, cnt = pool(p2, y2, dinv, batchp)
    x_log, x_sig, last = mlp(gsum, cnt, lin1_W, lin1_b, bn_gamma, bn_beta,
                             lin2_W, lin2_b)
    return (x_log, x_sig, last)


# speedup vs baseline: 18.8334x; 1.7507x over previous
import functools

import jax
import jax.numpy as jnp
from jax import lax
from jax.experimental import pallas as pl
from jax.experimental.pallas import tpu as pltpu
from jax.experimental.pallas import tpu_sc as plsc

F32 = jnp.float32
I32 = jnp.int32

NC = 2
NS = 16
NW = NC * NS

BM = 1024


def _pad_up(v, m):
    return (v + m - 1) // m * m


def _sc_mesh():
    return plsc.VectorSubcoreMesh(
        core_axis_name="c", subcore_axis_name="s", num_cores=NC, num_subcores=NS
    )


CH = 128


def _make_deg_kernel(npad, k):
    @functools.partial(
        pl.kernel,
        out_type=jax.ShapeDtypeStruct((NW, npad), F32),
        mesh=_sc_mesh(),
        scratch_types=[
            pltpu.VMEM((k, CH), I32),
            pltpu.VMEM((npad,), F32),
        ],
        compiler_params=pltpu.CompilerParams(needs_layout_passes=False),
    )
    def deg_kernel(dst_hbm, out_hbm, didx, dacc):
        cid = lax.axis_index("c")
        sid = lax.axis_index("s")
        wid = cid * NS + sid
        zero16 = jnp.zeros((16,), F32)

        def zbody(i, _):
            dacc[pl.ds(i * 16, 16)] = zero16
            return 0

        lax.fori_loop(0, npad // 16, zbody, 0)
        pltpu.sync_copy(dst_hbm.at[wid], didx)
        one16 = jnp.full((16,), 1.0, F32)

        def ebody(j, _):
            for i in range(CH // 16):
                idx = didx[j, pl.ds(i * 16, 16)]
                plsc.addupdate_scatter(dacc, [idx], one16)
            return 0

        lax.fori_loop(0, k, ebody, 0)
        pltpu.sync_copy(dacc, out_hbm.at[wid])

    return deg_kernel


def _make_agg_kernel(npad, k):
    rows_per_sub = npad // NS

    @functools.partial(
        pl.kernel,
        out_type=jax.ShapeDtypeStruct((NC, npad, 128), F32),
        mesh=_sc_mesh(),
        scratch_types=[
            pltpu.VMEM((k, CH), I32),
            pltpu.VMEM((k, CH), I32),
            pltpu.VMEM((CH, 128), F32),
            pltpu.VMEM_SHARED((npad, 128), F32),
            pltpu.SemaphoreType.DMA,
        ],
    )
    def agg_kernel(y_hbm, src_hbm, dst_hbm, zeros_hbm, out_hbm,
                   sidx, didx, rows, acc, sem):
        cid = lax.axis_index("c")
        sid = lax.axis_index("s")
        wid = cid * NS + sid
        pltpu.sync_copy(zeros_hbm, acc.at[pl.ds(sid * rows_per_sub, rows_per_sub)])
        pltpu.sync_copy(src_hbm.at[wid], sidx)
        pltpu.sync_copy(dst_hbm.at[wid], didx)
        plsc.subcore_barrier()

        def chunk(j, _):
            pltpu.async_copy(y_hbm.at[sidx.at[j]], rows, sem).wait()
            pltpu.sync_copy(rows, acc.at[didx.at[j]], add=True)
            return 0

        lax.fori_loop(0, k, chunk, 0)
        plsc.subcore_barrier()
        pltpu.sync_copy(
            acc.at[pl.ds(sid * rows_per_sub, rows_per_sub)],
            out_hbm.at[cid, pl.ds(sid * rows_per_sub, rows_per_sub)],
        )

    return agg_kernel


def _mm1_body(degp_ref, x_ref, w_ref, y_ref, dinv_ref):
    deg = jnp.sum(degp_ref[...], axis=0) + 1.0
    dinv = lax.rsqrt(deg)
    xw = jnp.dot(x_ref[...], w_ref[...], preferred_element_type=F32)
    y_ref[...] = xw * dinv[:, None]
    dinv_ref[...] = dinv


def _mmc_body(p_ref, y_ref, dinv_ref, w_ref, o_ref):
    dinv = dinv_ref[...][:, None]
    h = jnp.maximum((p_ref[0] + p_ref[1] + y_ref[...]) * dinv, 0.0)
    xw = jnp.dot(h, w_ref[...], preferred_element_type=F32)
    o_ref[...] = xw * dinv


def _pool_body(p_ref, y_ref, dinv_ref, batch_ref, gsum_ref, cnt_ref, *, G):
    i = pl.program_id(0)
    dinv = dinv_ref[...][:, None]
    h = jnp.maximum((p_ref[0] + p_ref[1] + y_ref[...]) * dinv, 0.0)
    b = batch_ref[...]
    onehot = (b[:, None] == lax.broadcasted_iota(I32, (1, G), 1)).astype(F32)
    gs = lax.dot_general(onehot, h, (((0,), (0,)), ((), ())),
                         preferred_element_type=F32,
                         precision=lax.Precision.HIGHEST)
    ones = jnp.ones((h.shape[0], 128), F32)
    cs = lax.dot_general(onehot, ones, (((0,), (0,)), ((), ())),
                         preferred_element_type=F32,
                         precision=lax.Precision.HIGHEST)

    @pl.when(i == 0)
    def _():
        gsum_ref[...] = gs
        cnt_ref[...] = cs

    @pl.when(i != 0)
    def _():
        gsum_ref[...] += gs
        cnt_ref[...] += cs


def _mlp_body(gsum_ref, cnt_ref, w1_ref, b1_ref, gam_ref, bet_ref, w2_ref, b2_ref,
              xlog_ref, xsig_ref, last_ref):
    cnt = jnp.maximum(cnt_ref[...], 1.0)
    g = gsum_ref[...] / cnt
    m = jnp.dot(g, w1_ref[...], preferred_element_type=F32) + b1_ref[...][None, :]
    mu = jnp.mean(m, axis=0, keepdims=True)
    var = jnp.mean((m - mu) * (m - mu), axis=0, keepdims=True)
    m = (m - mu) / jnp.sqrt(var + 1e-5) * gam_ref[...][None, :] + bet_ref[...][None, :]
    m = jnp.maximum(m, 0.0)
    out = jnp.dot(m, w2_ref[...], preferred_element_type=F32) + b2_ref[...][None, :]
    mx = jnp.max(out, axis=1, keepdims=True)
    lse = mx + jnp.log(jnp.sum(jnp.exp(out - mx), axis=1, keepdims=True))
    xlog_ref[...] = out - lse
    xsig_ref[...] = 1.0 / (1.0 + jnp.exp(-out))
    last_ref[...] = out


def kernel(x, edge_index, edge_weight, batch, W0, W1, W2,
           lin1_W, lin1_b, bn_gamma, bn_beta, lin2_W, lin2_b):
    del edge_weight
    n, d = x.shape
    e = edge_index.shape[1]
    h_dim = W0.shape[1]
    o_dim = lin2_W.shape[1]
    g_num = 128

    npad = _pad_up(n + 1, BM)
    ept = _pad_up(-(-e // NW), 8 * CH)
    k = ept // CH
    epad = NW * ept

    src = edge_index[0]
    dst = edge_index[1]
    pad_idx = (n + jnp.arange(epad - e, dtype=I32) % (npad - n)).astype(I32)
    srcp = jnp.concatenate([src, pad_idx]).reshape(NW, k, CH)
    dstp = jnp.concatenate([dst, pad_idx]).reshape(NW, k, CH)
    xp = jnp.pad(x, ((0, npad - n), (0, 0)))
    batchp = jnp.concatenate([batch, jnp.full((npad - n,), g_num, I32)])
    zeros2d = jnp.zeros((npad // NS, h_dim), F32)

    deg_call = _make_deg_kernel(npad, k)
    agg_call = _make_agg_kernel(npad, k)

    grid = npad // BM
    mm1 = pl.pallas_call(
        _mm1_body,
        grid=(grid,),
        in_specs=[
            pl.BlockSpec((NW, BM), lambda i: (0, i)),
            pl.BlockSpec((BM, d), lambda i: (i, 0)),
            pl.BlockSpec((d, h_dim), lambda i: (0, 0)),
        ],
        out_specs=[
            pl.BlockSpec((BM, h_dim), lambda i: (i, 0)),
            pl.BlockSpec((BM,), lambda i: (i,)),
        ],
        out_shape=[
            jax.ShapeDtypeStruct((npad, h_dim), F32),
            jax.ShapeDtypeStruct((npad,), F32),
        ],
    )
    mmc = pl.pallas_call(
        _mmc_body,
        grid=(grid,),
        in_specs=[
            pl.BlockSpec((NC, BM, h_dim), lambda i: (0, i, 0)),
            pl.BlockSpec((BM, h_dim), lambda i: (i, 0)),
            pl.BlockSpec((BM,), lambda i: (i,)),
            pl.BlockSpec((h_dim, h_dim), lambda i: (0, 0)),
        ],
        out_specs=pl.BlockSpec((BM, h_dim), lambda i: (i, 0)),
        out_shape=jax.ShapeDtypeStruct((npad, h_dim), F32),
    )
    pool = pl.pallas_call(
        functools.partial(_pool_body, G=g_num),
        grid=(grid,),
        in_specs=[
            pl.BlockSpec((NC, BM, h_dim), lambda i: (0, i, 0)),
            pl.BlockSpec((BM, h_dim), lambda i: (i, 0)),
            pl.BlockSpec((BM,), lambda i: (i,)),
            pl.BlockSpec((BM,), lambda i: (i,)),
        ],
        out_specs=[
            pl.BlockSpec((g_num, h_dim), lambda i: (0, 0)),
            pl.BlockSpec((g_num, 128), lambda i: (0, 0)),
        ],
        out_shape=[
            jax.ShapeDtypeStruct((g_num, h_dim), F32),
            jax.ShapeDtypeStruct((g_num, 128), F32),
        ],
    )
    mlp = pl.pallas_call(
        _mlp_body,
        out_shape=[
            jax.ShapeDtypeStruct((g_num, o_dim), F32),
            jax.ShapeDtypeStruct((g_num, o_dim), F32),
            jax.ShapeDtypeStruct((g_num, o_dim), F32),
        ],
    )

    degp = deg_call(dstp)
    y0, dinv = mm1(degp, xp, W0)
    p0 = agg_call(y0, srcp, dstp, zeros2d)
    y1 = mmc(p0, y0, dinv, W1)
    p1 = agg_call(y1, srcp, dstp, zeros2d)
    y2 = mmc(p1, y1, dinv, W2)
    p2 = agg_call(y2, srcp, dstp, zeros2d)
    gsum, cnt = pool(p2, y2, dinv, batchp)
    x_log, x_sig, last = mlp(gsum, cnt, lin1_W, lin1_b, bn_gamma, bn_beta,
                             lin2_W, lin2_b)
    return (x_log, x_sig, last)
